# baseline (speedup 1.0000x reference)
import functools

import jax
import jax.numpy as jnp
from jax import lax
from jax.experimental import pallas as pl
from jax.experimental.pallas import tpu as pltpu
from jax.experimental.pallas import tpu_sc as plsc

_B = 1024
_K = 8
_N = 8192

_NC = 2
_NS = 16
_NW = _NC * _NS

_NH = 2
_FH = _N // _NH
_RPW = _B // (_NW // _NH)
_RG = 4
_NG = _RPW // _RG
_NSLOT = 8
_L = 16
_CH = _FH // _L


def _group_splats(cond_v, row0, nrows):
    splats = []
    for pair in range(nrows // 2):
        v = cond_v[pl.ds((row0 + 2 * pair) * _K, 2 * _K)]
        for r in range(2):
            splats.append([jnp.full((_L,), v[r * _K + k], jnp.float32)
                           for k in range(_K)])
    return splats


def _sc_body(cond_hbm, w_hbm, b_hbm, out_hbm, cond_v, wr_v, wt_v, b_v, row_v,
             *sems):
    wid = lax.axis_index("s") * _NC + lax.axis_index("c")
    half = wid % _NH
    rowbase = (wid // _NH) * _RPW
    foff = half * _FH

    pltpu.sync_copy(w_hbm.at[pl.ds(foff * _K, _FH * _K)], wr_v)
    pltpu.sync_copy(b_hbm.at[pl.ds(half * _FH, _FH)], b_v)
    pltpu.sync_copy(cond_hbm.at[pl.ds(rowbase * _K, _RPW * _K)], cond_v)

    lanes = lax.iota(jnp.int32, _L)
    pattern = (lanes & 7) * _FH + (lanes >> 3)

    def tbody(jp, idx):
        v = wr_v[pl.ds(jp * _L, _L)]
        plsc.store_scatter(wt_v, [idx], v)
        return idx + 2

    lax.fori_loop(0, _FH * _K // _L, tbody, pattern, unroll=False)

    handles = [None] * _NSLOT
    for g in range(_NG):
        rows = [g * _RG + r for r in range(_RG)]
        slots = [row % _NSLOT for row in rows]
        cs = _group_splats(cond_v, g * _RG, _RG)
        for s in slots:
            if handles[s] is not None:
                handles[s].wait()
                handles[s] = None

        def body(j, carry, cs=cs, slots=slots):
            o = j * _L
            bv = b_v[pl.ds(o, _L)]
            wv = [wt_v[pl.ds(k * _FH + o, _L)] for k in range(_K)]
            for r in range(_RG):
                c = cs[r]
                a0 = bv + c[0] * wv[0]
                a0 = a0 + c[1] * wv[1]
                a0 = a0 + c[2] * wv[2]
                a0 = a0 + c[3] * wv[3]
                a1 = c[4] * wv[4] + c[5] * wv[5]
                a1 = a1 + c[6] * wv[6]
                a1 = a1 + c[7] * wv[7]
                row_v[pl.ds(slots[r] * _FH + o, _L)] = a0 + a1
            return carry

        lax.fori_loop(0, _CH, body, 0, unroll=False)

        for r in range(_RG):
            dst = (rowbase + rows[r]) * _N + foff
            handles[slots[r]] = pltpu.async_copy(
                row_v.at[pl.ds(slots[r] * _FH, _FH)],
                out_hbm.at[pl.ds(dst, _FH)],
                sems[slots[r]])

    for s in range(_NSLOT):
        if handles[s] is not None:
            handles[s].wait()


@functools.partial(jax.jit, static_argnames=())
def kernel(condition, W, b):
    run = pl.kernel(
        _sc_body,
        mesh=plsc.VectorSubcoreMesh(core_axis_name="c", subcore_axis_name="s"),
        compiler_params=pltpu.CompilerParams(needs_layout_passes=False),
        out_type=jax.ShapeDtypeStruct((_B * _N,), jnp.float32),
        scratch_types=(
            [
                pltpu.VMEM((_RPW * _K,), jnp.float32),
                pltpu.VMEM((_FH * _K,), jnp.float32),
                pltpu.VMEM((_K * _FH,), jnp.float32),
                pltpu.VMEM((_FH,), jnp.float32),
                pltpu.VMEM((_NSLOT * _FH,), jnp.float32)
            ] + [pltpu.SemaphoreType.DMA] * _NSLOT
        ),
    )
    out = run(condition.reshape(-1), W.reshape(-1), b)
    return out.reshape(_B, 32, 16, 16)


# speedup vs baseline: 2.8642x; 2.8642x over previous
import functools

import jax
import jax.numpy as jnp
from jax import lax
from jax.experimental import pallas as pl
from jax.experimental.pallas import tpu as pltpu
from jax.experimental.pallas import tpu_sc as plsc

_B = 1024
_K = 8
_N = 8192

_NC = 2
_NS = 16
_NW = _NC * _NS

_CH = 32
_HH = 16
_WW = 16
_L = 16
_JPW = _N // _NW
_HSZ = _WW * _B
_NM = _B // _L


def _sc_body(cond_hbm, w_hbm, b_hbm, out_hbm, condr_v, condt_v, w_v, b_v,
             slot_v, sem0, sem1):
    wid = lax.axis_index("s") * _NC + lax.axis_index("c")
    c = wid

    pltpu.sync_copy(cond_hbm, condr_v)
    pltpu.sync_copy(w_hbm.at[pl.ds(c * _JPW * _K, _JPW * _K)], w_v)
    pltpu.sync_copy(b_hbm.at[pl.ds(c * _JPW, _JPW)], b_v)

    lanes = lax.iota(jnp.int32, _L)
    pattern = (lanes & 7) * _B + (lanes >> 3)

    def tbody(m, idx):
        v = condr_v[pl.ds(m * _L, _L)]
        plsc.store_scatter(condt_v, [idx], v)
        return idx + 2

    lax.fori_loop(0, _B * _K // _L, tbody, pattern, unroll=False)

    sems = (sem0, sem1)
    handles = [None, None]
    for h in range(_HH):
        slot = h & 1
        if handles[slot] is not None:
            handles[slot].wait()
            handles[slot] = None
        bias16 = b_v[pl.ds(h * _WW, _L)]
        for p in range(8):
            w_hi = p >> 2
            w_lo0 = (p & 3) * 2
            wl = h * _WW + w_hi * 8 + w_lo0
            wpair = w_v[pl.ds(wl * _K, 2 * _K)]
            ws = [[jnp.full((_L,), wpair[jj * _K + k], jnp.float32)
                   for k in range(_K)] for jj in range(2)]
            bs = [jnp.full((_L,), bias16[w_hi * 8 + w_lo0 + jj], jnp.float32)
                  for jj in range(2)]
            base = slot * _HSZ + w_hi * 8192 + w_lo0 * 128

            def body(m, carry, ws=ws, bs=bs, base=base):
                cv = [condt_v[pl.ds(k * _B + m * _L, _L)] for k in range(_K)]
                addr = base + ((m >> 3) << 10) + ((m & 7) << 4)
                for jj in range(2):
                    w8 = ws[jj]
                    a0 = bs[jj] + w8[0] * cv[0]
                    a0 = a0 + w8[1] * cv[1]
                    a0 = a0 + w8[2] * cv[2]
                    a0 = a0 + w8[3] * cv[3]
                    a1 = w8[4] * cv[4] + w8[5] * cv[5]
                    a1 = a1 + w8[6] * cv[6]
                    a1 = a1 + w8[7] * cv[7]
                    slot_v[pl.ds(addr + jj * 128, _L)] = a0 + a1
                return carry

            lax.fori_loop(0, _NM, body, 0, unroll=False)
        handles[slot] = pltpu.async_copy(
            slot_v.at[pl.ds(slot * _HSZ, _HSZ)],
            out_hbm.at[pl.ds((c * _HH + h) * _HSZ, _HSZ)],
            sems[slot])

    for slot in range(2):
        if handles[slot] is not None:
            handles[slot].wait()


@functools.partial(jax.jit, static_argnames=())
def kernel(condition, W, b):
    run = pl.kernel(
        _sc_body,
        mesh=plsc.VectorSubcoreMesh(core_axis_name="c", subcore_axis_name="s"),
        compiler_params=pltpu.CompilerParams(needs_layout_passes=False),
        out_type=jax.ShapeDtypeStruct((_B * _N,), jnp.float32),
        scratch_types=(
            [
                pltpu.VMEM((_B * _K,), jnp.float32),
                pltpu.VMEM((_K * _B,), jnp.float32),
                pltpu.VMEM((_JPW * _K,), jnp.float32),
                pltpu.VMEM((_JPW,), jnp.float32),
                pltpu.VMEM((2 * _HSZ,), jnp.float32),
            ] + [pltpu.SemaphoreType.DMA] * 2
        ),
    )
    out = run(condition.reshape(-1), W.reshape(-1), b)
    y6 = out.reshape(_CH, _HH, 2, 8, 8, 128)
    return y6.transpose(3, 5, 0, 1, 2, 4).reshape(_B, _CH, _HH, _WW)


# speedup vs baseline: 20.5693x; 7.1816x over previous
import functools

import jax
import jax.numpy as jnp
from jax.experimental import pallas as pl
from jax.experimental.pallas import tpu as pltpu

_B = 1024
_K = 8
_N = 8192
_BJ = 1024


def _mm_kernel(wt_ref, ct_ref, b_ref, o_ref):
    wt = wt_ref[...]
    ct = ct_ref[...]
    acc = jax.lax.dot_general(wt, ct, (((0,), (0,)), ((), ())),
                              preferred_element_type=jnp.float32)
    o_ref[...] = acc + b_ref[...]


@functools.partial(jax.jit, static_argnames=())
def kernel(condition, W, b):
    yt = pl.pallas_call(
        _mm_kernel,
        grid=(_N // _BJ,),
        in_specs=[
            pl.BlockSpec((_K, _BJ), lambda i: (0, i)),
            pl.BlockSpec((_K, _B), lambda i: (0, 0)),
            pl.BlockSpec((_BJ, 1), lambda i: (i, 0)),
        ],
        out_specs=pl.BlockSpec((_BJ, _B), lambda i: (i, 0)),
        out_shape=jax.ShapeDtypeStruct((_N, _B), jnp.float32),
    )(W.T, condition.T, b.reshape(_N, 1))
    return yt.reshape(32, 16, 16, _B).transpose(3, 0, 1, 2)


# speedup vs baseline: 21.6790x; 1.0539x over previous
import functools

import jax
import jax.numpy as jnp
from jax.experimental import pallas as pl
from jax.experimental.pallas import tpu as pltpu

_B = 1024
_K = 8
_N = 8192
_BJ = 2048


def _mm_kernel(wt_ref, ct_ref, b_ref, o_ref):
    wt = wt_ref[...]
    ct = ct_ref[...]
    acc = jax.lax.dot_general(wt, ct, (((0,), (0,)), ((), ())),
                              preferred_element_type=jnp.float32)
    o_ref[...] = acc + b_ref[...]


@functools.partial(jax.jit, static_argnames=())
def kernel(condition, W, b):
    yt = pl.pallas_call(
        _mm_kernel,
        grid=(_N // _BJ,),
        in_specs=[
            pl.BlockSpec((_K, _BJ), lambda i: (0, i)),
            pl.BlockSpec((_K, _B), lambda i: (0, 0)),
            pl.BlockSpec((_BJ, 1), lambda i: (i, 0)),
        ],
        out_specs=pl.BlockSpec((_BJ, _B), lambda i: (i, 0)),
        out_shape=jax.ShapeDtypeStruct((_N, _B), jnp.float32),
    )(W.T, condition.T, b.reshape(_N, 1))
    return yt.reshape(32, 16, 16, _B).transpose(3, 0, 1, 2)
